# baseline (speedup 1.0000x reference)
import jax
import jax.numpy as jnp
from jax.experimental import pallas as pl

K = 100
C = 1024
HW = 1024
KH = 50


def _branch(ft, rn2_col, vcol, vrow, m_col, relu_cwf, fc1w, fc1b, fc2w, fc2b,
            tri):
    gt = vrow > vcol
    tie = (vrow == vcol) & tri
    rank_col = jnp.sum(jnp.where(gt | tie, 1, 0), axis=1, keepdims=True)
    kio = jax.lax.broadcasted_iota(jnp.int32, (HW, K), 1)
    pt = jnp.where(rank_col == kio, 1.0, 0.0)

    x_row = jax.lax.dot_general(relu_cwf, pt, (((0,), (0,)), ((), ())),
                                preferred_element_type=jnp.float32)
    h_row = jax.lax.dot_general(x_row, fc1w, (((1,), (1,)), ((), ())),
                                preferred_element_type=jnp.float32) + fc1b
    y_row = jax.lax.dot_general(h_row, fc2w, (((1,), (1,)), ((), ())),
                                preferred_element_type=jnp.float32) + fc2b
    gate_row = jax.nn.sigmoid(y_row)

    protos = jax.lax.dot_general(pt, ft, (((0,), (0,)), ((), ())),
                                 preferred_element_type=jnp.float32)
    rawt = jax.lax.dot_general(ft, protos, (((1,), (1,)), ((), ())),
                               preferred_element_type=jnp.float32)

    pn2_row = jax.lax.dot_general(rn2_col, pt, (((0,), (0,)), ((), ())),
                                  preferred_element_type=jnp.float32)
    pn_row = jnp.sqrt(gate_row * gate_row * pn2_row + 1e-12)
    on_col = jnp.sqrt(rn2_col * m_col + 1e-12)

    att = (rawt * m_col) * gate_row / (on_col * pn_row + 1e-8)
    att = jnp.maximum(att, 0.0)
    return jnp.max(att, axis=1, keepdims=True)


def _body(ft_ref, soft_ref,
          cw_f_ref, fc1w_f_ref, fc1b_f_ref, fc2w_f_ref, fc2b_f_ref,
          cw_b_ref, fc1w_b_ref, fc1b_b_ref, fc2w_b_ref, fc2b_b_ref,
          out_ref):
    ft = ft_ref[0]
    soft = jnp.reshape(soft_ref[0], (2, HW))
    s0r = soft[0:1, :]
    s1r = soft[1:2, :]
    s0c = jnp.reshape(s0r, (HW, 1))
    s1c = jnp.reshape(s1r, (HW, 1))

    ii = jax.lax.broadcasted_iota(jnp.int32, (HW, HW), 1)
    jj = jax.lax.broadcasted_iota(jnp.int32, (HW, HW), 0)
    tri = ii < jj

    fg_col = jnp.where(s1c > s0c, 1.0, 0.0)
    bg_col = 1.0 - fg_col

    rn2_col = jnp.sum(ft * ft, axis=1, keepdims=True)

    cw2 = jnp.concatenate([jnp.reshape(cw_f_ref[...], (1, C)),
                           jnp.reshape(cw_b_ref[...], (1, C))], axis=0)
    cwf2 = jax.lax.dot_general(ft, cw2, (((1,), (1,)), ((), ())),
                               preferred_element_type=jnp.float32)
    relu_cwf2 = jnp.maximum(cwf2, 0.0)

    fore = _branch(ft, rn2_col, s1c, s1r, bg_col, relu_cwf2[:, 0:1],
                   fc1w_f_ref[...], jnp.reshape(fc1b_f_ref[...], (1, KH)),
                   fc2w_f_ref[...], jnp.reshape(fc2b_f_ref[...], (1, K)),
                   tri)
    back = _branch(ft, rn2_col, s0c, s0r, fg_col, relu_cwf2[:, 1:2],
                   fc1w_b_ref[...], jnp.reshape(fc1b_b_ref[...], (1, KH)),
                   fc2w_b_ref[...], jnp.reshape(fc2b_b_ref[...], (1, K)),
                   tri)

    out_ref[0] = ft * (1.0 + s1c - back + fore)


def kernel(feats, soft_mask, conv_w_f, fc1_w_f, fc1_b_f, fc2_w_f, fc2_b_f,
           conv_w_b, fc1_w_b, fc1_b_b, fc2_w_b, fc2_b_b):
    b, c, h, w = feats.shape
    hw = h * w
    ft3 = jnp.transpose(feats, (0, 2, 3, 1)).reshape(b, hw, c)

    args = (
        ft3, soft_mask,
        conv_w_f, fc1_w_f, fc1_b_f, fc2_w_f, fc2_b_f,
        conv_w_b, fc1_w_b, fc1_b_b, fc2_w_b, fc2_b_b,
    )

    def fixed(shape):
        return pl.BlockSpec(shape, lambda i: (0,) * len(shape))

    out_t = pl.pallas_call(
        _body,
        grid=(b,),
        in_specs=[
            pl.BlockSpec((1, hw, c), lambda i: (i, 0, 0)),
            pl.BlockSpec((1, 2, h, w), lambda i: (i, 0, 0, 0)),
            fixed((c,)),
            fixed((KH, K)), fixed((KH,)), fixed((K, KH)), fixed((K,)),
            fixed((c,)),
            fixed((KH, K)), fixed((KH,)), fixed((K, KH)), fixed((K,)),
        ],
        out_specs=pl.BlockSpec((1, hw, c), lambda i: (i, 0, 0)),
        out_shape=jax.ShapeDtypeStruct((b, hw, c), jnp.float32),
    )(*args)
    return jnp.transpose(out_t.reshape(b, h, w, c), (0, 3, 1, 2))
